# baseline (speedup 1.0000x reference)
import functools

import jax
import jax.numpy as jnp
from jax import lax
from jax.experimental import pallas as pl
from jax.experimental.pallas import tpu as pltpu
from jax.experimental.pallas import tpu_sc as plsc

_E = 8
_K = 2
_BK = 128
_INV_SQRT2 = 0.7071067811865476


def _router_body(x_ref, wr_ref, idx_ref, w_ref):
    x = x_ref[...]
    wr = wr_ref[...]
    logits = lax.dot_general(x, wr, (((1,), (1,)), ((), ())),
                             preferred_element_type=jnp.float32)
    m = jnp.max(logits, axis=1, keepdims=True)
    p = jnp.exp(logits - m)
    p = p / jnp.sum(p, axis=1, keepdims=True)
    ii = lax.broadcasted_iota(jnp.int32, p.shape, 1)
    m1 = jnp.max(p, axis=1, keepdims=True)
    a1 = jnp.min(jnp.where(p == m1, ii, _E), axis=1, keepdims=True)
    p2 = jnp.where(ii == a1, -1.0, p)
    m2 = jnp.max(p2, axis=1, keepdims=True)
    a2 = jnp.min(jnp.where(p2 == m2, ii, _E), axis=1, keepdims=True)
    s = jnp.maximum(m1 + m2, 1e-12)
    idx_ref[...] = jnp.where(ii == 0, a1, jnp.where(ii == 1, a2, 0))
    w_ref[...] = jnp.where(ii == 0, m1 / s, jnp.where(ii == 1, m2 / s, 0.0))


def _router(tokens, Wr, interpret=False):
    T, D = tokens.shape
    TB = 1024
    return pl.pallas_call(
        _router_body,
        grid=(T // TB,),
        in_specs=[pl.BlockSpec((TB, D), lambda i: (i, 0)),
                  pl.BlockSpec((_E, D), lambda i: (0, 0))],
        out_specs=[pl.BlockSpec((TB, _E), lambda i: (i, 0)),
                   pl.BlockSpec((TB, _E), lambda i: (i, 0))],
        out_shape=[jax.ShapeDtypeStruct((T, _E), jnp.int32),
                   jax.ShapeDtypeStruct((T, _E), jnp.float32)],
        interpret=interpret,
    )(tokens, Wr)


def _ffn_body(te_ref, x_ref, w1_ref, b1_ref, w2_ref, b2_ref, ws_ref, y_ref):
    del te_ref
    x = x_ref[...].astype(jnp.bfloat16)
    h = lax.dot_general(x, w1_ref[0].astype(jnp.bfloat16),
                        (((1,), (1,)), ((), ())),
                        preferred_element_type=jnp.float32)
    h = h + b1_ref[0]
    h = 0.5 * h * (1.0 + lax.erf(h * _INV_SQRT2))
    y = lax.dot_general(h.astype(jnp.bfloat16), w2_ref[0].astype(jnp.bfloat16),
                        (((1,), (1,)), ((), ())),
                        preferred_element_type=jnp.float32)
    y = y + b2_ref[0]
    y_ref[...] = y * ws_ref[...]


def _grouped_ffn(tile_expert, x_sorted, W1, b1, W2, b2, ws, interpret=False):
    P, D = x_sorted.shape
    NT = P // _BK
    grid_spec = pltpu.PrefetchScalarGridSpec(
        num_scalar_prefetch=1,
        grid=(NT,),
        in_specs=[
            pl.BlockSpec((_BK, D), lambda i, te: (i, 0)),
            pl.BlockSpec((1, D, D), lambda i, te: (te[i], 0, 0)),
            pl.BlockSpec((1, 1, D), lambda i, te: (te[i], 0, 0)),
            pl.BlockSpec((1, D, D), lambda i, te: (te[i], 0, 0)),
            pl.BlockSpec((1, 1, D), lambda i, te: (te[i], 0, 0)),
            pl.BlockSpec((_BK, 1), lambda i, te: (i, 0)),
        ],
        out_specs=pl.BlockSpec((_BK, D), lambda i, te: (i, 0)),
    )
    return pl.pallas_call(
        _ffn_body,
        grid_spec=grid_spec,
        out_shape=jax.ShapeDtypeStruct((P, D), jnp.float32),
        interpret=interpret,
    )(tile_expert, x_sorted, W1, b1[:, None, :], W2, b2[:, None, :], ws)


def _dispatch(eidx, wtop, T, P):
    A = T * _K
    ea = eidx[:, :_K].reshape(-1)
    wa = wtop[:, :_K].reshape(-1)
    onehot = (ea[:, None] == jnp.arange(_E)[None, :]).astype(jnp.int32)
    csum = jnp.cumsum(onehot, axis=0)
    counts = csum[-1]
    rank = jnp.sum(onehot * csum, axis=1) - 1
    tiles = (counts + _BK - 1) // _BK
    cum_tiles = jnp.cumsum(tiles)
    pad_start = ((cum_tiles - tiles) * _BK).astype(jnp.int32)
    pos_a = jnp.sum(onehot * pad_start[None, :], axis=1) + rank
    row_token = jnp.zeros((P,), jnp.int32).at[pos_a].set(
        (jnp.arange(A, dtype=jnp.int32) // _K))
    ws = jnp.zeros((P,), jnp.float32).at[pos_a].set(wa)
    pos2 = pos_a.reshape(T, _K)
    ti = jnp.arange(P // _BK, dtype=jnp.int32)
    tile_expert = jnp.minimum(
        jnp.sum(ti[:, None] >= cum_tiles[None, :], axis=1), _E - 1
    ).astype(jnp.int32)
    return row_token, ws, pos2[:, 0], pos2[:, 1], tile_expert


def kernel(input_embeddings, Wr, W1, b1, W2, b2):
    Bs, Ss, D = input_embeddings.shape
    T = Bs * Ss
    P = (T * _K // _BK + _E) * _BK
    tokens = input_embeddings.reshape(T, D)

    eidx, wtop = _router(tokens, Wr)
    row_token, ws, pos0, pos1, tile_expert = _dispatch(eidx, wtop, T, P)

    x_sorted = jnp.take(tokens, row_token, axis=0)
    y = _grouped_ffn(tile_expert, x_sorted, W1, b1, W2, b2, ws[:, None])
    out = tokens + jnp.take(y, pos0, axis=0) + jnp.take(y, pos1, axis=0)
    return out.reshape(Bs, Ss, D)


# speedup vs baseline: 1.0802x; 1.0802x over previous
import functools

import jax
import jax.numpy as jnp
from jax import lax
from jax.experimental import pallas as pl
from jax.experimental.pallas import tpu as pltpu
from jax.experimental.pallas import tpu_sc as plsc

_E = 8
_K = 2
_BK = 128
_INV_SQRT2 = 0.7071067811865476


def _router_body(x_ref, wr_ref, idx_ref, w_ref, rank_ref, aux_ref, run_ref):
    i = pl.program_id(0)
    nb = pl.num_programs(0)

    @pl.when(i == 0)
    def _init():
        run_ref[...] = jnp.zeros_like(run_ref)

    x = x_ref[...]
    wr = wr_ref[...]
    logits = lax.dot_general(x, wr, (((1,), (1,)), ((), ())),
                             preferred_element_type=jnp.float32)
    m = jnp.max(logits, axis=1, keepdims=True)
    p = jnp.exp(logits - m)
    p = p / jnp.sum(p, axis=1, keepdims=True)
    ii = lax.broadcasted_iota(jnp.int32, p.shape, 1)
    m1 = jnp.max(p, axis=1, keepdims=True)
    a1 = jnp.min(jnp.where(p == m1, ii, _E), axis=1, keepdims=True)
    p2 = jnp.where(ii == a1, -1.0, p)
    m2 = jnp.max(p2, axis=1, keepdims=True)
    a2 = jnp.min(jnp.where(p2 == m2, ii, _E), axis=1, keepdims=True)
    s = jnp.maximum(m1 + m2, 1e-12)

    TB = p.shape[0]
    oh = jnp.logical_or(ii == a1, ii == a2).astype(jnp.float32)
    ri = lax.broadcasted_iota(jnp.int32, (TB, TB), 0)
    ci = lax.broadcasted_iota(jnp.int32, (TB, TB), 1)
    lt = (ri > ci).astype(jnp.float32)
    excl = lax.dot_general(lt, oh, (((1,), (0,)), ((), ())),
                           preferred_element_type=jnp.float32)
    run = run_ref[0:1, 0:_E]
    base = run + excl
    rank0 = jnp.sum(jnp.where(ii == a1, base, 0.0), axis=1, keepdims=True)
    rank1 = jnp.sum(jnp.where(ii == a2, base, 0.0), axis=1, keepdims=True)
    run_new = run + excl[TB - 1:TB, :] + oh[TB - 1:TB, :]
    run_ref[0:1, 0:_E] = run_new

    i2 = lax.broadcasted_iota(jnp.int32, (TB, _K), 1)
    idx_ref[...] = jnp.where(i2 == 0, a1, a2)
    w_ref[...] = jnp.where(i2 == 0, m1 / s, m2 / s)
    rank_ref[...] = jnp.where(i2 == 0, rank0, rank1).astype(jnp.int32)

    @pl.when(i == nb - 1)
    def _fin():
        cnt = run_new
        tiles = jnp.floor((cnt + (_BK - 1)) * (1.0 / _BK))
        r8 = lax.broadcasted_iota(jnp.int32, (_E, _E), 0)
        c8 = lax.broadcasted_iota(jnp.int32, (_E, _E), 1)
        ut = (r8 <= c8).astype(jnp.float32)
        cumt = lax.dot_general(tiles, ut, (((1,), (0,)), ((), ())),
                               preferred_element_type=jnp.float32)
        pad_start = (cumt - tiles) * _BK
        aux_ref[...] = jnp.concatenate(
            [pad_start.astype(jnp.int32), cumt.astype(jnp.int32),
             jnp.zeros((1, 128 - 2 * _E), jnp.int32)], axis=1)


def _router(tokens, Wr, interpret=False):
    T, D = tokens.shape
    TB = 1024
    return pl.pallas_call(
        _router_body,
        grid=(T // TB,),
        in_specs=[pl.BlockSpec((TB, D), lambda i: (i, 0)),
                  pl.BlockSpec((_E, D), lambda i: (0, 0))],
        out_specs=[pl.BlockSpec((TB, _K), lambda i: (i, 0)),
                   pl.BlockSpec((TB, _K), lambda i: (i, 0)),
                   pl.BlockSpec((TB, _K), lambda i: (i, 0)),
                   pl.BlockSpec((1, 128), lambda i: (0, 0))],
        out_shape=[jax.ShapeDtypeStruct((T, _K), jnp.int32),
                   jax.ShapeDtypeStruct((T, _K), jnp.float32),
                   jax.ShapeDtypeStruct((T, _K), jnp.int32),
                   jax.ShapeDtypeStruct((1, 128), jnp.int32)],
        scratch_shapes=[pltpu.VMEM((1, 128), jnp.float32)],
        interpret=interpret,
    )(tokens, Wr)


def _ffn_body(te_ref, x_ref, w1_ref, b1_ref, w2_ref, b2_ref, y_ref):
    del te_ref
    x = x_ref[...]
    h = lax.dot_general(x, w1_ref[0], (((1,), (1,)), ((), ())),
                        preferred_element_type=jnp.float32)
    h = h + b1_ref[0]
    h = 0.5 * h * (1.0 + lax.erf(h * _INV_SQRT2))
    y = lax.dot_general(h, w2_ref[0], (((1,), (1,)), ((), ())),
                        preferred_element_type=jnp.float32)
    y_ref[...] = y + b2_ref[0]


def _grouped_ffn(tile_expert, x_sorted, W1, b1, W2, b2, interpret=False):
    P, D = x_sorted.shape
    NT = P // _BK
    grid_spec = pltpu.PrefetchScalarGridSpec(
        num_scalar_prefetch=1,
        grid=(NT,),
        in_specs=[
            pl.BlockSpec((_BK, D), lambda i, te: (i, 0)),
            pl.BlockSpec((1, D, D), lambda i, te: (te[i], 0, 0)),
            pl.BlockSpec((1, 1, D), lambda i, te: (te[i], 0, 0)),
            pl.BlockSpec((1, D, D), lambda i, te: (te[i], 0, 0)),
            pl.BlockSpec((1, 1, D), lambda i, te: (te[i], 0, 0)),
        ],
        out_specs=pl.BlockSpec((_BK, D), lambda i, te: (i, 0)),
    )
    return pl.pallas_call(
        _ffn_body,
        grid_spec=grid_spec,
        out_shape=jax.ShapeDtypeStruct((P, D), jnp.float32),
        interpret=interpret,
    )(tile_expert, x_sorted, W1, b1[:, None, :], W2, b2[:, None, :])


def kernel(input_embeddings, Wr, W1, b1, W2, b2):
    Bs, Ss, D = input_embeddings.shape
    T = Bs * Ss
    A = T * _K
    P = (A // _BK + _E) * _BK
    tokens = input_embeddings.reshape(T, D)

    idx01, w01, rank01, aux = _router(tokens, Wr)
    ea = idx01.reshape(A)
    pad_start = aux[0, :_E]
    cum_tiles = aux[0, _E:2 * _E]
    pos01 = jnp.take(pad_start, ea) + rank01.reshape(A)
    row_token = jnp.zeros((P,), jnp.int32).at[pos01].set(
        jnp.arange(A, dtype=jnp.int32) // _K)
    ti = jnp.arange(P // _BK, dtype=jnp.int32)
    tile_expert = jnp.minimum(
        jnp.sum(ti[:, None] >= cum_tiles[None, :], axis=1), _E - 1
    ).astype(jnp.int32)

    x_sorted = jnp.take(tokens, row_token, axis=0)
    y = _grouped_ffn(tile_expert, x_sorted, W1, b1, W2, b2)
    pos2 = pos01.reshape(T, _K)
    out = (tokens
           + w01[:, 0:1] * jnp.take(y, pos2[:, 0], axis=0)
           + w01[:, 1:2] * jnp.take(y, pos2[:, 1], axis=0))
    return out.reshape(Bs, Ss, D)
